# baseline (speedup 1.0000x reference)
import functools

import jax
import jax.numpy as jnp
from jax import lax
from jax.experimental import pallas as pl
from jax.experimental.pallas import tpu as pltpu
from jax.experimental.pallas import tpu_sc as plsc

_THRESHOLD = 0.01
_NC = 2
_NW = 32
_BLK = 512
_NBLK = 1953
_ABLK = 62
_X_UNITS = 896
_BU = 512
_BUNITS = 50


def _phase_a(w_t, m_t, tail_rows):
    vocab = w_t.shape[1]
    dim = w_t.shape[0]
    tail = vocab - _NBLK * _BLK
    mesh = plsc.VectorSubcoreMesh(core_axis_name="c", subcore_axis_name="s")

    @functools.partial(
        pl.kernel,
        mesh=mesh,
        compiler_params=pltpu.CompilerParams(use_tc_tiling_on_sc=True,
                                             needs_layout_passes=False),
        out_type=jax.ShapeDtypeStruct((vocab * dim,), jnp.float32),
        scratch_types=[
            pltpu.VMEM((2, dim, _BLK + 1), jnp.float32),
            pltpu.VMEM((2, dim, _BLK + 1), jnp.float32),
            pltpu.VMEM((2, _BLK * dim), jnp.float32),
            [pltpu.SemaphoreType.DMA] * 2,
            [pltpu.SemaphoreType.DMA] * 2,
        ],
    )
    def k(w_hbm, m_hbm, tl_hbm, tab_hbm, wv, mv, ov, sem_in, sem_out):
        wid = lax.axis_index("s") * _NC + lax.axis_index("c")
        iota = lax.iota(jnp.int32, 16)

        def blk_of(k_it):
            return pl.multiple_of(
                lax.rem(wid + k_it * _NW, _NBLK) * _BLK, _BLK)

        def fire_in(k_it, par):
            v0 = blk_of(k_it)
            pltpu.async_copy(w_hbm.at[:, pl.ds(v0, _BLK)],
                             wv.at[par, :, pl.ds(0, _BLK)], sem_in[par])
            pltpu.async_copy(m_hbm.at[:, pl.ds(v0, _BLK)],
                             mv.at[par, :, pl.ds(0, _BLK)], sem_in[par])

        def wait_in(k_it, par):
            v0 = blk_of(k_it)
            pltpu.make_async_copy(w_hbm.at[:, pl.ds(v0, _BLK)],
                                  wv.at[par, :, pl.ds(0, _BLK)],
                                  sem_in[par]).wait()
            pltpu.make_async_copy(m_hbm.at[:, pl.ds(v0, _BLK)],
                                  mv.at[par, :, pl.ds(0, _BLK)],
                                  sem_in[par]).wait()

        def wait_out(k_it, par):
            v0 = blk_of(k_it)
            pltpu.make_async_copy(
                ov.at[par], tab_hbm.at[pl.ds(v0 * dim, _BLK * dim)],
                sem_out[par]).wait()

        @pl.when(wid == 0)
        def _():
            pltpu.sync_copy(tl_hbm, ov.at[0, pl.ds(0, tail * dim)])
            pltpu.sync_copy(ov.at[0, pl.ds(0, tail * dim)],
                            tab_hbm.at[pl.ds(_NBLK * _BLK * dim,
                                             tail * dim)])

        fire_in(0, 0)
        fire_in(1, 1)

        def half_body(k_it, par):
            wait_in(k_it, par)

            @pl.when(k_it >= 2)
            def _():
                wait_out(k_it - 2, par)

            def mrow(r, _):
                @plsc.parallel_loop(0, _BLK // 16, unroll=8)
                def seg(s):
                    off = s * 16
                    w = wv[par, r, pl.ds(off, 16)]
                    m = mv[par, r, pl.ds(off, 16)]
                    wv[par, r, pl.ds(off, 16)] = jnp.where(
                        m > _THRESHOLD, w, 0.0)

                return 0

            lax.fori_loop(0, dim, mrow, 0)

            @plsc.parallel_loop(0, _BLK, unroll=32)
            def col(c):
                ci = jnp.full((16,), c, jnp.int32)
                ov[par, pl.ds(c * dim, dim)] = plsc.load_gather(
                    wv.at[par], [iota, ci])
            v0 = blk_of(k_it)
            pltpu.async_copy(ov.at[par],
                             tab_hbm.at[pl.ds(v0 * dim, _BLK * dim)],
                             sem_out[par])

            @pl.when(k_it < _ABLK - 2)
            def _():
                fire_in(k_it + 2, par)

        def blk_pair(k2, _):
            half_body(k2 * 2, 0)
            half_body(k2 * 2 + 1, 1)
            return 0

        lax.fori_loop(0, _ABLK // 2, blk_pair, 0)
        wait_out(_ABLK - 2, 0)
        wait_out(_ABLK - 1, 1)

    return k(w_t, m_t, tail_rows)


def _phase_b(idx_flat, tab, n_j, n_b):
    mesh = plsc.VectorSubcoreMesh(core_axis_name="c", subcore_axis_name="s")
    dim = tab.shape[1]
    per_w = _BUNITS * _BU
    upj = n_b // _BU

    @functools.partial(
        pl.kernel,
        mesh=mesh,
        compiler_params=pltpu.CompilerParams(use_tc_tiling_on_sc=False,
                                             needs_layout_passes=False),
        out_type=jax.ShapeDtypeStruct((n_j, 2, n_b // 128, 8, 128),
                                      jnp.float32),
        scratch_types=[
            pltpu.VMEM((per_w,), jnp.int32),
            pltpu.VMEM((2, _BU, dim), jnp.float32),
            pltpu.VMEM((2, dim, _BU + 1), jnp.float32),
            [pltpu.SemaphoreType.DMA] * 2,
            [pltpu.SemaphoreType.DMA] * 2,
        ],
    )
    def k(idx_hbm, tab_hbm, out_hbm, iv, gv, tv, sem_g, sem_o):
        wid = lax.axis_index("s") * _NC + lax.axis_index("c")
        iota = lax.iota(jnp.int32, 16)
        u0 = wid * _BUNITS

        pltpu.sync_copy(idx_hbm.at[pl.ds(wid * per_w, per_w)], iv)

        def fire_gather(u, par):
            pltpu.async_copy(tab_hbm.at[iv.at[pl.ds(u * _BU, _BU)]],
                             gv.at[par], sem_g[par])

        def wait_gather(u, par):
            pltpu.make_async_copy(tab_hbm.at[iv.at[pl.ds(u * _BU, _BU)]],
                                  gv.at[par], sem_g[par]).wait()

        def out_copies(u, par, fire):
            uu = u0 + u
            j = uu // upj
            q4 = lax.rem(uu, upj)
            for g in range(2):
                for cq in range(4):
                    src = tv.at[par, pl.ds(g * 8, 8), pl.ds(cq * 128, 128)]
                    dst = out_hbm.at[j, g, q4 * 4 + cq]
                    if fire:
                        pltpu.async_copy(src, dst, sem_o[par])
                    else:
                        pltpu.make_async_copy(src, dst, sem_o[par]).wait()

        fire_gather(0, 0)
        fire_gather(1, 1)

        def unit_half(u, par):
            wait_gather(u, par)

            @pl.when(u >= 2)
            def _():
                out_copies(u - 2, par, fire=False)

            @plsc.parallel_loop(0, _BU, unroll=8)
            def row(b):
                v = gv[par, b, :]
                plsc.store_scatter(tv.at[par],
                                   [iota, jnp.full((16,), b, jnp.int32)], v)

            out_copies(u, par, fire=True)

            @pl.when(u < _BUNITS - 2)
            def _():
                fire_gather(u + 2, par)

        def unit_pair(u2, _):
            unit_half(u2 * 2, 0)
            unit_half(u2 * 2 + 1, 1)
            return 0

        lax.fori_loop(0, _BUNITS // 2, unit_pair, 0)
        out_copies(_BUNITS - 2, 0, fire=False)
        out_copies(_BUNITS - 1, 1, fire=False)

    return k(idx_flat, tab)


def kernel(x, weight, mask_real):
    n_b0, n_j = x.shape
    vocab, dim = weight.shape
    x_t = x.astype(jnp.int32).T
    v0 = _NBLK * _BLK
    tail_rows = ((mask_real[v0:] > _THRESHOLD) * weight[v0:]).reshape(-1)
    idx_flat = x_t.reshape(-1)
    tab_flat = _phase_a(weight.T, mask_real.T, tail_rows)
    tab = tab_flat.reshape(vocab, dim)
    out5 = _phase_b(idx_flat, tab, n_j, n_b0)
    return out5.transpose(2, 4, 0, 1, 3).reshape(n_b0, n_j, dim)


# speedup vs baseline: 1.0222x; 1.0222x over previous
import functools

import jax
import jax.numpy as jnp
from jax import lax
from jax.experimental import pallas as pl
from jax.experimental.pallas import tpu as pltpu
from jax.experimental.pallas import tpu_sc as plsc

_THRESHOLD = 0.01
_NC = 2
_NW = 32
_BLK = 512
_NBLK = 1953
_ABLK = 62
_BU = 512
_BUNITS = 50


def _phase_a(w_t, m_t, tail_rows):
    vocab = w_t.shape[1]
    dim = w_t.shape[0]
    tail = vocab - _NBLK * _BLK
    mesh = plsc.VectorSubcoreMesh(core_axis_name="c", subcore_axis_name="s")

    @functools.partial(
        pl.kernel,
        mesh=mesh,
        compiler_params=pltpu.CompilerParams(use_tc_tiling_on_sc=True,
                                             needs_layout_passes=False),
        out_type=jax.ShapeDtypeStruct((vocab * dim,), jnp.float32),
        scratch_types=[
            pltpu.VMEM((2, dim, _BLK + 1), jnp.float32),
            pltpu.VMEM((2, dim, _BLK + 1), jnp.float32),
            pltpu.VMEM((2, _BLK * dim), jnp.float32),
            [pltpu.SemaphoreType.DMA] * 2,
            [pltpu.SemaphoreType.DMA] * 2,
        ],
    )
    def k(w_hbm, m_hbm, tl_hbm, tab_hbm, wv, mv, ov, sem_in, sem_out):
        wid = lax.axis_index("s") * _NC + lax.axis_index("c")
        iota = lax.iota(jnp.int32, 16)

        def blk_of(k_it):
            return pl.multiple_of(
                lax.rem(wid + k_it * _NW, _NBLK) * _BLK, _BLK)

        def fire_in(k_it, par):
            v0 = blk_of(k_it)
            pltpu.async_copy(w_hbm.at[:, pl.ds(v0, _BLK)],
                             wv.at[par, :, pl.ds(0, _BLK)], sem_in[par])
            pltpu.async_copy(m_hbm.at[:, pl.ds(v0, _BLK)],
                             mv.at[par, :, pl.ds(0, _BLK)], sem_in[par])

        def wait_in(k_it, par):
            v0 = blk_of(k_it)
            pltpu.make_async_copy(w_hbm.at[:, pl.ds(v0, _BLK)],
                                  wv.at[par, :, pl.ds(0, _BLK)],
                                  sem_in[par]).wait()
            pltpu.make_async_copy(m_hbm.at[:, pl.ds(v0, _BLK)],
                                  mv.at[par, :, pl.ds(0, _BLK)],
                                  sem_in[par]).wait()

        def wait_out(k_it, par):
            v0 = blk_of(k_it)
            pltpu.make_async_copy(
                ov.at[par], tab_hbm.at[pl.ds(v0 * dim, _BLK * dim)],
                sem_out[par]).wait()

        @pl.when(wid == 0)
        def _():
            pltpu.sync_copy(tl_hbm, ov.at[0, pl.ds(0, tail * dim)])
            pltpu.sync_copy(ov.at[0, pl.ds(0, tail * dim)],
                            tab_hbm.at[pl.ds(_NBLK * _BLK * dim,
                                             tail * dim)])

        fire_in(0, 0)
        fire_in(1, 1)

        def half_body(k_it, par):
            wait_in(k_it, par)

            @pl.when(k_it >= 2)
            def _():
                wait_out(k_it - 2, par)

            def mrow(r, _):
                @plsc.parallel_loop(0, _BLK // 16, unroll=8)
                def seg(s):
                    off = s * 16
                    w = wv[par, r, pl.ds(off, 16)]
                    m = mv[par, r, pl.ds(off, 16)]
                    wv[par, r, pl.ds(off, 16)] = jnp.where(
                        m > _THRESHOLD, w, 0.0)

                return 0

            lax.fori_loop(0, dim, mrow, 0)

            @plsc.parallel_loop(0, _BLK, unroll=16)
            def col(c):
                ci = jnp.full((16,), c, jnp.int32)
                ov[par, pl.ds(c * dim, dim)] = plsc.load_gather(
                    wv.at[par], [iota, ci])
            v0 = blk_of(k_it)
            pltpu.async_copy(ov.at[par],
                             tab_hbm.at[pl.ds(v0 * dim, _BLK * dim)],
                             sem_out[par])

            @pl.when(k_it < _ABLK - 2)
            def _():
                fire_in(k_it + 2, par)

        def blk_pair(k2, _):
            half_body(k2 * 2, 0)
            half_body(k2 * 2 + 1, 1)
            return 0

        lax.fori_loop(0, _ABLK // 2, blk_pair, 0)
        wait_out(_ABLK - 2, 0)
        wait_out(_ABLK - 1, 1)

    return k(w_t, m_t, tail_rows)


def _phase_b(idx_flat, tab, n_j, n_b):
    mesh = plsc.VectorSubcoreMesh(core_axis_name="c", subcore_axis_name="s")
    dim = tab.shape[1]
    per_w = _BUNITS * _BU
    upj = n_b // _BU

    @functools.partial(
        pl.kernel,
        mesh=mesh,
        compiler_params=pltpu.CompilerParams(use_tc_tiling_on_sc=False,
                                             needs_layout_passes=False),
        out_type=jax.ShapeDtypeStruct((n_j, 2, n_b // 128, 8, 128),
                                      jnp.float32),
        scratch_types=[
            pltpu.VMEM((per_w,), jnp.int32),
            pltpu.VMEM((2, _BU, dim), jnp.float32),
            pltpu.VMEM((2, dim, _BU + 1), jnp.float32),
            [pltpu.SemaphoreType.DMA] * 2,
            [pltpu.SemaphoreType.DMA] * 2,
        ],
    )
    def k(idx_hbm, tab_hbm, out_hbm, iv, gv, tv, sem_g, sem_o):
        wid = lax.axis_index("s") * _NC + lax.axis_index("c")
        iota = lax.iota(jnp.int32, 16)
        u0 = wid * _BUNITS

        pltpu.sync_copy(idx_hbm.at[pl.ds(wid * per_w, per_w)], iv)

        def fire_gather(u, par):
            pltpu.async_copy(tab_hbm.at[iv.at[pl.ds(u * _BU, _BU)]],
                             gv.at[par], sem_g[par])

        def wait_gather(u, par):
            pltpu.make_async_copy(tab_hbm.at[iv.at[pl.ds(u * _BU, _BU)]],
                                  gv.at[par], sem_g[par]).wait()

        def out_copies(u, par, fire):
            uu = u0 + u
            j = uu // upj
            q4 = lax.rem(uu, upj)
            for g in range(2):
                for cq in range(4):
                    src = tv.at[par, pl.ds(g * 8, 8), pl.ds(cq * 128, 128)]
                    dst = out_hbm.at[j, g, q4 * 4 + cq]
                    if fire:
                        pltpu.async_copy(src, dst, sem_o[par])
                    else:
                        pltpu.make_async_copy(src, dst, sem_o[par]).wait()

        fire_gather(0, 0)
        fire_gather(1, 1)

        def unit_half(u, par):
            wait_gather(u, par)

            @pl.when(u >= 2)
            def _():
                out_copies(u - 2, par, fire=False)

            @plsc.parallel_loop(0, _BU, unroll=4)
            def row(b):
                v = gv[par, b, :]
                plsc.store_scatter(tv.at[par],
                                   [iota, jnp.full((16,), b, jnp.int32)], v)

            out_copies(u, par, fire=True)

            @pl.when(u < _BUNITS - 2)
            def _():
                fire_gather(u + 2, par)

        def unit_pair(u2, _):
            unit_half(u2 * 2, 0)
            unit_half(u2 * 2 + 1, 1)
            return 0

        lax.fori_loop(0, _BUNITS // 2, unit_pair, 0)
        out_copies(_BUNITS - 2, 0, fire=False)
        out_copies(_BUNITS - 1, 1, fire=False)

    return k(idx_flat, tab)


def kernel(x, weight, mask_real):
    n_b0, n_j = x.shape
    vocab, dim = weight.shape
    x_t = x.astype(jnp.int32).T
    v0 = _NBLK * _BLK
    tail_rows = ((mask_real[v0:] > _THRESHOLD) * weight[v0:]).reshape(-1)
    idx_flat = x_t.reshape(-1)
    tab_flat = _phase_a(weight.T, mask_real.T, tail_rows)
    tab = tab_flat.reshape(vocab, dim)
    out5 = _phase_b(idx_flat, tab, n_j, n_b0)
    return out5.transpose(2, 4, 0, 1, 3).reshape(n_b0, n_j, dim)
